# Initial kernel scaffold; baseline (speedup 1.0000x reference)
import jax
import jax.numpy as jnp
from jax.experimental import pallas as pl


def kernel(x, edge_index, edge_weight, att, W_cz, b_cz, W_lz, b_lz, W_cr, b_cr, W_lr, b_lr, W_ch, b_ch, W_lh, b_lh, W_lin, b_lin):
    raise NotImplementedError("write your pallas kernel here")


# speedup vs baseline: 200.4160x; 200.4160x over previous
import functools

import jax
import jax.numpy as jnp
from jax import lax
from jax.experimental import pallas as pl
from jax.experimental.pallas import tpu as pltpu
from jax.experimental.pallas import tpu_sc as plsc

N = 50000
E = 800000
PERIODS = 12
HID = 100

NC = 2
NS = 16
NW = NC * NS
CHUNK = 128
EPAD = 802816
G = EPAD // CHUNK
GPT = G // NW
NPAD = 50176
STRIPE = NPAD // NS


def _deg_body(idx2, w2d, degp, didx_v, wrow_v, zb, acc):
    c = lax.axis_index("c")
    s = lax.axis_index("s")
    wid = s * NC + c

    def zero_body(i, carry):
        zb[pl.ds(i * 16, 16)] = jnp.zeros((16,), jnp.float32)
        return carry

    lax.fori_loop(0, STRIPE // 16, zero_body, 0)
    pltpu.sync_copy(zb, acc.at[pl.ds(s * STRIPE, STRIPE)])
    plsc.subcore_barrier()

    def body(g, carry):
        pltpu.sync_copy(idx2.at[g, 1], didx_v)
        pltpu.sync_copy(w2d.at[g], wrow_v)
        pltpu.sync_copy(wrow_v, acc.at[didx_v], add=True)
        return carry

    lax.fori_loop(wid * GPT, (wid + 1) * GPT, body, 0)
    plsc.subcore_barrier()
    pltpu.sync_copy(acc.at[pl.ds(s * STRIPE, STRIPE)], zb)
    pltpu.sync_copy(zb, degp.at[pl.ds(c * NPAD + s * STRIPE, STRIPE)])


def _scatter_body(idx2, w2d, tab, sp, sidx_v, didx_v, wrow_v, rows_v, zb,
                  acc, sem):
    c = lax.axis_index("c")
    s = lax.axis_index("s")
    wid = s * NC + c

    def zero_body(i, carry):
        zb[i, :] = jnp.zeros((16,), jnp.float32)
        return carry

    lax.fori_loop(0, STRIPE, zero_body, 0)
    pltpu.sync_copy(zb, acc.at[pl.ds(s * STRIPE, STRIPE)])
    plsc.subcore_barrier()

    def body(g, carry):
        pltpu.sync_copy(idx2.at[g, 0], sidx_v)
        pltpu.sync_copy(idx2.at[g, 1], didx_v)
        pltpu.sync_copy(w2d.at[g], wrow_v)
        pltpu.async_copy(tab.at[sidx_v], rows_v, sem).wait()
        for b in range(CHUNK // 16):
            w16 = wrow_v[pl.ds(b * 16, 16)]
            for l in range(16):
                e = b * 16 + l
                rows_v[e, :] = rows_v[e, :] * w16[l]
        pltpu.sync_copy(rows_v, acc.at[didx_v], add=True)
        return carry

    lax.fori_loop(wid * GPT, (wid + 1) * GPT, body, 0)
    plsc.subcore_barrier()
    pltpu.sync_copy(acc.at[pl.ds(s * STRIPE, STRIPE)], zb)
    pltpu.sync_copy(zb, sp.at[pl.ds(c * NPAD + s * STRIPE, STRIPE)])


_deg_call = functools.partial(
    pl.kernel,
    out_type=jax.ShapeDtypeStruct((NC * NPAD,), jnp.float32),
    mesh=plsc.VectorSubcoreMesh(core_axis_name="c", subcore_axis_name="s"),
    compiler_params=pltpu.CompilerParams(use_tc_tiling_on_sc=False),
    scratch_types=[
        pltpu.VMEM((CHUNK,), jnp.int32),
        pltpu.VMEM((CHUNK,), jnp.float32),
        pltpu.VMEM((STRIPE,), jnp.float32),
        pltpu.VMEM_SHARED((NPAD,), jnp.float32),
    ],
)(_deg_body)

_scatter_call = functools.partial(
    pl.kernel,
    out_type=jax.ShapeDtypeStruct((NC * NPAD, 16), jnp.float32),
    mesh=plsc.VectorSubcoreMesh(core_axis_name="c", subcore_axis_name="s"),
    compiler_params=pltpu.CompilerParams(use_tc_tiling_on_sc=False),
    scratch_types=[
        pltpu.VMEM((CHUNK,), jnp.int32),
        pltpu.VMEM((CHUNK,), jnp.int32),
        pltpu.VMEM((CHUNK,), jnp.float32),
        pltpu.VMEM((CHUNK, 16), jnp.float32),
        pltpu.VMEM((STRIPE, 16), jnp.float32),
        pltpu.VMEM_SHARED((NPAD, 16), jnp.float32),
        pltpu.SemaphoreType.DMA,
    ],
)(_scatter_body)


BN = 2000


def _gate_body(s0, s1, aux, att, Wcz, bcz, Wlz, blz, Wch, bch, Wlh, blh,
               wlin, blin, out):
    a = jnp.exp(att[...] - jnp.max(att[...]))
    probs = a / jnp.sum(a)
    u_z = jnp.dot(Wcz[...], Wlz[...], preferred_element_type=jnp.float32)
    c_z = jnp.dot(bcz[...], Wlz[...], preferred_element_type=jnp.float32) \
        + blz[...]
    u_h = jnp.dot(Wch[...], Wlh[...], preferred_element_type=jnp.float32)
    c_h = jnp.dot(bch[...], Wlh[...], preferred_element_type=jnp.float32) \
        + blh[...]
    dinv = aux[:, 12:13]
    gall = dinv * (s0[...] + s1[...] + aux[...])
    acc = jnp.zeros((BN, HID), jnp.float32)
    for t in range(PERIODS):
        g = gall[:, t:t + 1]
        z = jax.nn.sigmoid(g * u_z + c_z)
        ht = jnp.tanh(g * u_h + c_h)
        acc = acc + probs[0, t] * (1.0 - z) * ht
    h = jnp.maximum(acc, 0.0)
    out[...] = jnp.sum(h * wlin[...], axis=1, keepdims=True) + blin[...]


def _gate_call(s0, s1, aux, att, Wcz, bcz, Wlz, blz, Wch, bch, Wlh, blh,
               wlin, blin):
    grid = (N // BN,)
    blk = lambda shape: pl.BlockSpec(shape, lambda i: (0,) * len(shape))
    return pl.pallas_call(
        _gate_body,
        grid=grid,
        in_specs=[
            pl.BlockSpec((BN, 16), lambda i: (i, 0)),
            pl.BlockSpec((BN, 16), lambda i: (i, 0)),
            pl.BlockSpec((BN, 16), lambda i: (i, 0)),
            blk((1, PERIODS)),
            blk((1, HID)),
            blk((1, HID)),
            blk((HID, HID)),
            blk((1, HID)),
            blk((1, HID)),
            blk((1, HID)),
            blk((HID, HID)),
            blk((1, HID)),
            blk((1, HID)),
            blk((1, 1)),
        ],
        out_specs=pl.BlockSpec((BN, 1), lambda i: (i, 0)),
        out_shape=jax.ShapeDtypeStruct((N, 1), jnp.float32),
    )(s0, s1, aux, att, Wcz, bcz, Wlz, blz, Wch, bch, Wlh, blh, wlin, blin)


def kernel(x, edge_index, edge_weight, att, W_cz, b_cz, W_lz, b_lz, W_cr,
           b_cr, W_lr, b_lr, W_ch, b_ch, W_lh, b_lh, W_lin, b_lin):
    src = edge_index[0].astype(jnp.int32)
    dst = edge_index[1].astype(jnp.int32)
    w = edge_weight.astype(jnp.float32)

    npadidx = (jnp.arange(EPAD - E, dtype=jnp.int32) * 173) % N
    src_p = jnp.concatenate([src, npadidx]).reshape(G, CHUNK)
    dst_p = jnp.concatenate([dst, npadidx]).reshape(G, CHUNK)
    w_p = jnp.concatenate(
        [w, jnp.zeros((EPAD - E,), jnp.float32)]).reshape(G, CHUNK)
    idx2 = jnp.stack([src_p, dst_p], axis=1)

    degp = _deg_call(idx2, w_p)
    deg = degp[:N] + degp[NPAD:NPAD + N] + 1.0
    dinv = lax.rsqrt(deg)

    aux = jnp.concatenate(
        [dinv[:, None] * x, dinv[:, None], jnp.zeros((N, 3), jnp.float32)],
        axis=1)

    sp = _scatter_call(idx2, w_p, aux)

    return _gate_call(
        sp[:N], sp[NPAD:NPAD + N], aux,
        att.reshape(1, PERIODS),
        W_cz.reshape(1, HID), b_cz.reshape(1, HID), W_lz[:HID],
        b_lz.reshape(1, HID),
        W_ch.reshape(1, HID), b_ch.reshape(1, HID), W_lh[:HID],
        b_lh.reshape(1, HID),
        W_lin.reshape(1, HID), b_lin.reshape(1, 1))


# speedup vs baseline: 354.5160x; 1.7689x over previous
import functools

import jax
import jax.numpy as jnp
from jax import lax
from jax.experimental import pallas as pl
from jax.experimental.pallas import tpu as pltpu
from jax.experimental.pallas import tpu_sc as plsc

N = 50000
E = 800000
PERIODS = 12
HID = 100

NC = 2
NS = 16
NW = NC * NS
CHUNK = 128
EPAD = 802816
G = EPAD // CHUNK
GPT = G // NW
NPAD = 50176
STRIPE = NPAD // NS


NBUF = 4


def _deg_body(idx2, w2d, degp, didx4, wrow4, zb, acc, se0, se1, se2, se3,
              ss0, ss1, ss2, ss3):
    c = lax.axis_index("c")
    s = lax.axis_index("s")
    wid = s * NC + c
    base = wid * GPT
    se = [se0, se1, se2, se3]
    ss = [ss0, ss1, ss2, ss3]

    def zero_body(i, carry):
        zb[pl.ds(i * 16, 16)] = jnp.zeros((16,), jnp.float32)
        return carry

    lax.fori_loop(0, STRIPE // 16, zero_body, 0)
    pltpu.sync_copy(zb, acc.at[pl.ds(s * STRIPE, STRIPE)])
    plsc.subcore_barrier()

    def issue_load(g, b):
        pltpu.async_copy(idx2.at[g, 1], didx4.at[b], se[b])
        pltpu.async_copy(w2d.at[g], wrow4.at[b], se[b])

    def wait_load(g, b):
        pltpu.make_async_copy(idx2.at[g, 1], didx4.at[b], se[b]).wait()
        pltpu.make_async_copy(w2d.at[g], wrow4.at[b], se[b]).wait()

    def issue_scat(b):
        pltpu.async_copy(wrow4.at[b], acc.at[didx4.at[b]], ss[b], add=True)

    def wait_scat(b):
        pltpu.make_async_copy(
            wrow4.at[b], acc.at[didx4.at[b]], ss[b]).wait()

    issue_load(base + 0, 0)
    issue_load(base + 1, 1)
    for q in (0, 1):
        wait_load(base + q, q)
        issue_scat(q)
        issue_load(base + q + 2, (q + 2) % NBUF)

    def body(k, carry):
        for boff in range(NBUF):
            g = base + 2 + k * NBUF + boff
            b = (2 + boff) % NBUF
            wait_load(g, b)
            issue_scat(b)
            b2 = (b + 2) % NBUF
            wait_scat(b2)
            issue_load(g + 2, b2)
        return carry

    lax.fori_loop(0, (GPT - 4) // NBUF, body, 0)

    for q in (GPT - 2, GPT - 1):
        b = q % NBUF
        wait_load(base + q, b)
        issue_scat(b)
        wait_scat((b + 2) % NBUF)
    wait_scat((GPT - 2) % NBUF)
    wait_scat((GPT - 1) % NBUF)

    plsc.subcore_barrier()
    pltpu.sync_copy(acc.at[pl.ds(s * STRIPE, STRIPE)], zb)
    pltpu.sync_copy(zb, degp.at[pl.ds(c * NPAD + s * STRIPE, STRIPE)])


def _scatter_body(idx2, w2d, tab, sp, sidx4, didx4, wrow4, rows4, zb,
                  acc, se0, se1, se2, se3, sg0, sg1, sg2, sg3,
                  ss0, ss1, ss2, ss3):
    c = lax.axis_index("c")
    s = lax.axis_index("s")
    wid = s * NC + c
    base = wid * GPT
    se = [se0, se1, se2, se3]
    sg = [sg0, sg1, sg2, sg3]
    ss = [ss0, ss1, ss2, ss3]

    def zero_body(i, carry):
        zb[i, :] = jnp.zeros((16,), jnp.float32)
        return carry

    lax.fori_loop(0, STRIPE, zero_body, 0)
    pltpu.sync_copy(zb, acc.at[pl.ds(s * STRIPE, STRIPE)])
    plsc.subcore_barrier()

    def issue_load(g, b):
        pltpu.async_copy(idx2.at[g, 0], sidx4.at[b], se[b])
        pltpu.async_copy(idx2.at[g, 1], didx4.at[b], se[b])
        pltpu.async_copy(w2d.at[g], wrow4.at[b], se[b])

    def wait_load(g, b):
        pltpu.make_async_copy(idx2.at[g, 0], sidx4.at[b], se[b]).wait()
        pltpu.make_async_copy(idx2.at[g, 1], didx4.at[b], se[b]).wait()
        pltpu.make_async_copy(w2d.at[g], wrow4.at[b], se[b]).wait()

    def issue_gather(b):
        pltpu.async_copy(tab.at[sidx4.at[b]], rows4.at[b], sg[b])

    def wait_gather(b):
        pltpu.make_async_copy(tab.at[sidx4.at[b]], rows4.at[b],
                              sg[b]).wait()

    def issue_scat(b):
        pltpu.async_copy(rows4.at[b], acc.at[didx4.at[b]], ss[b], add=True)

    def wait_scat(b):
        pltpu.make_async_copy(
            rows4.at[b], acc.at[didx4.at[b]], ss[b]).wait()

    def scale(b):
        for blk in range(CHUNK // 16):
            w16 = wrow4[b, pl.ds(blk * 16, 16)]
            for l in range(16):
                e = blk * 16 + l
                rows4[b, e, :] = rows4[b, e, :] * w16[l]

    def steady(g, b, first2, last2):
        wait_gather(b)
        scale(b)
        issue_scat(b)
        b1 = (b + 1) % NBUF
        if not last2:
            wait_load(g + 1, b1)
            issue_gather(b1)
        b2 = (b + 2) % NBUF
        if not first2:
            wait_scat(b2)
        if not last2:
            issue_load(g + 2, b2)

    issue_load(base + 0, 0)
    issue_load(base + 1, 1)
    wait_load(base + 0, 0)
    issue_gather(0)
    steady(base + 0, 0, True, False)
    steady(base + 1, 1, True, False)

    def body(k, carry):
        for boff in range(NBUF):
            g = base + 2 + k * NBUF + boff
            steady(g, (2 + boff) % NBUF, False, False)
        return carry

    lax.fori_loop(0, (GPT - 4) // NBUF, body, 0)

    for q in (GPT - 2, GPT - 1):
        b = q % NBUF
        if q == GPT - 2:
            wait_load(base + q + 1, (b + 1) % NBUF)
            issue_gather((b + 1) % NBUF)
            wait_gather(b)
        else:
            wait_gather(b)
        scale(b)
        issue_scat(b)
        wait_scat((b + 2) % NBUF)
    wait_scat((GPT - 2) % NBUF)
    wait_scat((GPT - 1) % NBUF)

    plsc.subcore_barrier()
    pltpu.sync_copy(acc.at[pl.ds(s * STRIPE, STRIPE)], zb)
    pltpu.sync_copy(zb, sp.at[pl.ds(c * NPAD + s * STRIPE, STRIPE)])


_deg_call = functools.partial(
    pl.kernel,
    out_type=jax.ShapeDtypeStruct((NC * NPAD,), jnp.float32),
    mesh=plsc.VectorSubcoreMesh(core_axis_name="c", subcore_axis_name="s"),
    compiler_params=pltpu.CompilerParams(use_tc_tiling_on_sc=False),
    scratch_types=[
        pltpu.VMEM((NBUF, CHUNK), jnp.int32),
        pltpu.VMEM((NBUF, CHUNK), jnp.float32),
        pltpu.VMEM((STRIPE,), jnp.float32),
        pltpu.VMEM_SHARED((NPAD,), jnp.float32),
    ] + [pltpu.SemaphoreType.DMA] * 8,
)(_deg_body)

_scatter_call = functools.partial(
    pl.kernel,
    out_type=jax.ShapeDtypeStruct((NC * NPAD, 16), jnp.float32),
    mesh=plsc.VectorSubcoreMesh(core_axis_name="c", subcore_axis_name="s"),
    compiler_params=pltpu.CompilerParams(use_tc_tiling_on_sc=False),
    scratch_types=[
        pltpu.VMEM((NBUF, CHUNK), jnp.int32),
        pltpu.VMEM((NBUF, CHUNK), jnp.int32),
        pltpu.VMEM((NBUF, CHUNK), jnp.float32),
        pltpu.VMEM((NBUF, CHUNK, 16), jnp.float32),
        pltpu.VMEM((STRIPE, 16), jnp.float32),
        pltpu.VMEM_SHARED((NPAD, 16), jnp.float32),
    ] + [pltpu.SemaphoreType.DMA] * 12,
)(_scatter_body)


BN = 2000


def _gate_body(s0, s1, aux, att, Wcz, bcz, Wlz, blz, Wch, bch, Wlh, blh,
               wlin, blin, out):
    a = jnp.exp(att[...] - jnp.max(att[...]))
    probs = a / jnp.sum(a)
    u_z = jnp.dot(Wcz[...], Wlz[...], preferred_element_type=jnp.float32)
    c_z = jnp.dot(bcz[...], Wlz[...], preferred_element_type=jnp.float32) \
        + blz[...]
    u_h = jnp.dot(Wch[...], Wlh[...], preferred_element_type=jnp.float32)
    c_h = jnp.dot(bch[...], Wlh[...], preferred_element_type=jnp.float32) \
        + blh[...]
    dinv = aux[:, 12:13]
    gall = dinv * (s0[...] + s1[...] + aux[...])
    acc = jnp.zeros((BN, HID), jnp.float32)
    for t in range(PERIODS):
        g = gall[:, t:t + 1]
        z = jax.nn.sigmoid(g * u_z + c_z)
        ht = jnp.tanh(g * u_h + c_h)
        acc = acc + probs[0, t] * (1.0 - z) * ht
    h = jnp.maximum(acc, 0.0)
    out[...] = jnp.sum(h * wlin[...], axis=1, keepdims=True) + blin[...]


def _gate_call(s0, s1, aux, att, Wcz, bcz, Wlz, blz, Wch, bch, Wlh, blh,
               wlin, blin):
    grid = (N // BN,)
    blk = lambda shape: pl.BlockSpec(shape, lambda i: (0,) * len(shape))
    return pl.pallas_call(
        _gate_body,
        grid=grid,
        in_specs=[
            pl.BlockSpec((BN, 16), lambda i: (i, 0)),
            pl.BlockSpec((BN, 16), lambda i: (i, 0)),
            pl.BlockSpec((BN, 16), lambda i: (i, 0)),
            blk((1, PERIODS)),
            blk((1, HID)),
            blk((1, HID)),
            blk((HID, HID)),
            blk((1, HID)),
            blk((1, HID)),
            blk((1, HID)),
            blk((HID, HID)),
            blk((1, HID)),
            blk((1, HID)),
            blk((1, 1)),
        ],
        out_specs=pl.BlockSpec((BN, 1), lambda i: (i, 0)),
        out_shape=jax.ShapeDtypeStruct((N, 1), jnp.float32),
    )(s0, s1, aux, att, Wcz, bcz, Wlz, blz, Wch, bch, Wlh, blh, wlin, blin)


def kernel(x, edge_index, edge_weight, att, W_cz, b_cz, W_lz, b_lz, W_cr,
           b_cr, W_lr, b_lr, W_ch, b_ch, W_lh, b_lh, W_lin, b_lin):
    src = edge_index[0].astype(jnp.int32)
    dst = edge_index[1].astype(jnp.int32)
    w = edge_weight.astype(jnp.float32)

    npadidx = (jnp.arange(EPAD - E, dtype=jnp.int32) * 173) % N
    src_p = jnp.concatenate([src, npadidx]).reshape(G, CHUNK)
    dst_p = jnp.concatenate([dst, npadidx]).reshape(G, CHUNK)
    w_p = jnp.concatenate(
        [w, jnp.zeros((EPAD - E,), jnp.float32)]).reshape(G, CHUNK)
    idx2 = jnp.stack([src_p, dst_p], axis=1)

    degp = _deg_call(idx2, w_p)
    deg = degp[:N] + degp[NPAD:NPAD + N] + 1.0
    dinv = lax.rsqrt(deg)

    aux = jnp.concatenate(
        [dinv[:, None] * x, dinv[:, None], jnp.zeros((N, 3), jnp.float32)],
        axis=1)

    sp = _scatter_call(idx2, w_p, aux)

    return _gate_call(
        sp[:N], sp[NPAD:NPAD + N], aux,
        att.reshape(1, PERIODS),
        W_cz.reshape(1, HID), b_cz.reshape(1, HID), W_lz[:HID],
        b_lz.reshape(1, HID),
        W_ch.reshape(1, HID), b_ch.reshape(1, HID), W_lh[:HID],
        b_lh.reshape(1, HID),
        W_lin.reshape(1, HID), b_lin.reshape(1, 1))


# speedup vs baseline: 438.7164x; 1.2375x over previous
import functools

import jax
import jax.numpy as jnp
from jax import lax
from jax.experimental import pallas as pl
from jax.experimental.pallas import tpu as pltpu
from jax.experimental.pallas import tpu_sc as plsc

N = 50000
E = 800000
PERIODS = 12
HID = 100

NC = 2
NS = 16
NW = NC * NS
CHUNK = 128
GPT = 200
G = NW * GPT
EPAD = G * CHUNK
NPAD = 50176
STRIPE = NPAD // NS


def _deg_body(dst2d, w2d, degp0, degp1, didx4, wrow4, zb, acc,
              se0, se1, se2, se3, ss0, ss1, ss2, ss3):
    c = lax.axis_index("c")
    s = lax.axis_index("s")
    wid = s * NC + c
    base = wid * GPT
    se = [se0, se1, se2, se3]
    ss = [ss0, ss1, ss2, ss3]
    DB = 4

    def zero_body(i, carry):
        zb[pl.ds(i * 16, 16)] = jnp.zeros((16,), jnp.float32)
        return carry

    lax.fori_loop(0, STRIPE // 16, zero_body, 0)
    pltpu.sync_copy(zb, acc.at[pl.ds(s * STRIPE, STRIPE)])
    plsc.subcore_barrier()

    def issue_load(g, b):
        pltpu.async_copy(dst2d.at[g], didx4.at[b], se[b])
        pltpu.async_copy(w2d.at[g], wrow4.at[b], se[b])

    def wait_load(g, b):
        pltpu.make_async_copy(dst2d.at[g], didx4.at[b], se[b]).wait()
        pltpu.make_async_copy(w2d.at[g], wrow4.at[b], se[b]).wait()

    def issue_scat(b):
        pltpu.async_copy(wrow4.at[b], acc.at[didx4.at[b]], ss[b], add=True)

    def wait_scat(b):
        pltpu.make_async_copy(
            wrow4.at[b], acc.at[didx4.at[b]], ss[b]).wait()

    issue_load(base + 0, 0)
    issue_load(base + 1, 1)
    for q in (0, 1):
        wait_load(base + q, q)
        issue_scat(q)
        issue_load(base + q + 2, (q + 2) % DB)

    def body(k, carry):
        for boff in range(DB):
            g = base + 2 + k * DB + boff
            b = (2 + boff) % DB
            wait_load(g, b)
            issue_scat(b)
            b2 = (b + 2) % DB
            wait_scat(b2)
            issue_load(g + 2, b2)
        return carry

    lax.fori_loop(0, (GPT - 4) // DB, body, 0)

    for q in (GPT - 2, GPT - 1):
        b = q % DB
        wait_load(base + q, b)
        issue_scat(b)
        wait_scat((b + 2) % DB)
    wait_scat((GPT - 2) % DB)
    wait_scat((GPT - 1) % DB)

    plsc.subcore_barrier()
    pltpu.sync_copy(acc.at[pl.ds(s * STRIPE, STRIPE)], zb)

    @pl.when(c == 0)
    def _():
        pltpu.sync_copy(zb, degp0.at[pl.ds(s * STRIPE, STRIPE)])

    @pl.when(c == 1)
    def _():
        pltpu.sync_copy(zb, degp1.at[pl.ds(s * STRIPE, STRIPE)])


NBUF = 8


def _scatter_body(src2d, dst2d, w2d, tab, sp0, sp1,
                  sidx8, didx8, wrow8, rows8, zb, acc, *sems):
    c = lax.axis_index("c")
    s = lax.axis_index("s")
    wid = s * NC + c
    base = wid * GPT
    se = sems[0:NBUF]
    sg = sems[NBUF:2 * NBUF]
    ss = sems[2 * NBUF:3 * NBUF]

    def zero_body(i, carry):
        zb[i, :] = jnp.zeros((16,), jnp.float32)
        return carry

    lax.fori_loop(0, STRIPE, zero_body, 0)
    pltpu.sync_copy(zb, acc.at[pl.ds(s * STRIPE, STRIPE)])
    plsc.subcore_barrier()

    def issue_load(g, b):
        pltpu.async_copy(src2d.at[g], sidx8.at[b], se[b])
        pltpu.async_copy(dst2d.at[g], didx8.at[b], se[b])
        pltpu.async_copy(w2d.at[g], wrow8.at[b], se[b])

    def wait_load(g, b):
        pltpu.make_async_copy(src2d.at[g], sidx8.at[b], se[b]).wait()
        pltpu.make_async_copy(dst2d.at[g], didx8.at[b], se[b]).wait()
        pltpu.make_async_copy(w2d.at[g], wrow8.at[b], se[b]).wait()

    def issue_gather(b):
        pltpu.async_copy(tab.at[sidx8.at[b]], rows8.at[b], sg[b])

    def wait_gather(b):
        pltpu.make_async_copy(tab.at[sidx8.at[b]], rows8.at[b],
                              sg[b]).wait()

    def issue_scat(b):
        pltpu.async_copy(rows8.at[b], acc.at[didx8.at[b]], ss[b], add=True)

    def wait_scat(b):
        pltpu.make_async_copy(
            rows8.at[b], acc.at[didx8.at[b]], ss[b]).wait()

    def scale(b):
        for blk in range(CHUNK // 16):
            w16 = wrow8[b, pl.ds(blk * 16, 16)]
            for l in range(16):
                e = blk * 16 + l
                rows8[b, e, :] = rows8[b, e, :] * w16[l]

    def step(g, sl, gather2, scatwait, load4):
        wait_gather(sl)
        scale(sl)
        issue_scat(sl)
        if gather2:
            sl2 = (sl + 2) % NBUF
            wait_load(g + 2, sl2)
            issue_gather(sl2)
        if scatwait:
            wait_scat((sl + 4) % NBUF)
        if load4:
            issue_load(g + 4, (sl + 4) % NBUF)

    for q in range(4):
        issue_load(base + q, q)
    for q in (0, 1):
        wait_load(base + q, q)
        issue_gather(q)
    for q in range(4):
        step(base + q, q, True, False, True)

    def body(k, carry):
        for boff in range(NBUF):
            g = base + 4 + k * NBUF + boff
            step(g, (4 + boff) % NBUF, True, True, True)
        return carry

    lax.fori_loop(0, (GPT - 8) // NBUF, body, 0)

    for q in (GPT - 4, GPT - 3):
        step(base + q, q % NBUF, True, True, False)
    for q in (GPT - 2, GPT - 1):
        step(base + q, q % NBUF, False, True, False)
    for q in range(GPT - 4, GPT):
        wait_scat(q % NBUF)

    plsc.subcore_barrier()
    pltpu.sync_copy(acc.at[pl.ds(s * STRIPE, STRIPE)], zb)

    @pl.when(c == 0)
    def _():
        pltpu.sync_copy(zb, sp0.at[pl.ds(s * STRIPE, STRIPE)])

    @pl.when(c == 1)
    def _():
        pltpu.sync_copy(zb, sp1.at[pl.ds(s * STRIPE, STRIPE)])


_deg_call = functools.partial(
    pl.kernel,
    out_type=(jax.ShapeDtypeStruct((NPAD,), jnp.float32),
              jax.ShapeDtypeStruct((NPAD,), jnp.float32)),
    mesh=plsc.VectorSubcoreMesh(core_axis_name="c", subcore_axis_name="s"),
    compiler_params=pltpu.CompilerParams(use_tc_tiling_on_sc=False),
    scratch_types=[
        pltpu.VMEM((4, CHUNK), jnp.int32),
        pltpu.VMEM((4, CHUNK), jnp.float32),
        pltpu.VMEM((STRIPE,), jnp.float32),
        pltpu.VMEM_SHARED((NPAD,), jnp.float32),
    ] + [pltpu.SemaphoreType.DMA] * 8,
)(_deg_body)

_scatter_call = functools.partial(
    pl.kernel,
    out_type=(jax.ShapeDtypeStruct((NPAD, 16), jnp.float32),
              jax.ShapeDtypeStruct((NPAD, 16), jnp.float32)),
    mesh=plsc.VectorSubcoreMesh(core_axis_name="c", subcore_axis_name="s"),
    compiler_params=pltpu.CompilerParams(use_tc_tiling_on_sc=False),
    scratch_types=[
        pltpu.VMEM((NBUF, CHUNK), jnp.int32),
        pltpu.VMEM((NBUF, CHUNK), jnp.int32),
        pltpu.VMEM((NBUF, CHUNK), jnp.float32),
        pltpu.VMEM((NBUF, CHUNK, 16), jnp.float32),
        pltpu.VMEM((STRIPE, 16), jnp.float32),
        pltpu.VMEM_SHARED((NPAD, 16), jnp.float32),
    ] + [pltpu.SemaphoreType.DMA] * 24,
)(_scatter_body)


BN = 2000


def _gate_body(s0, s1, aux, att, Wcz, bcz, Wlz, blz, Wch, bch, Wlh, blh,
               wlin, blin, out):
    a = jnp.exp(att[...] - jnp.max(att[...]))
    probs = a / jnp.sum(a)
    u_z = jnp.dot(Wcz[...], Wlz[...], preferred_element_type=jnp.float32)
    c_z = jnp.dot(bcz[...], Wlz[...], preferred_element_type=jnp.float32) \
        + blz[...]
    u_h = jnp.dot(Wch[...], Wlh[...], preferred_element_type=jnp.float32)
    c_h = jnp.dot(bch[...], Wlh[...], preferred_element_type=jnp.float32) \
        + blh[...]
    dinv = aux[:, 12:13]
    gall = dinv * (s0[...] + s1[...] + aux[...])
    acc = jnp.zeros((BN, HID), jnp.float32)
    for t in range(PERIODS):
        g = gall[:, t:t + 1]
        z = jax.nn.sigmoid(g * u_z + c_z)
        ht = jnp.tanh(g * u_h + c_h)
        acc = acc + probs[0, t] * (1.0 - z) * ht
    h = jnp.maximum(acc, 0.0)
    out[...] = jnp.sum(h * wlin[...], axis=1, keepdims=True) + blin[...]


def _gate_call(s0, s1, aux, att, Wcz, bcz, Wlz, blz, Wch, bch, Wlh, blh,
               wlin, blin):
    grid = (N // BN,)
    blk = lambda shape: pl.BlockSpec(shape, lambda i: (0,) * len(shape))
    return pl.pallas_call(
        _gate_body,
        grid=grid,
        in_specs=[
            pl.BlockSpec((BN, 16), lambda i: (i, 0)),
            pl.BlockSpec((BN, 16), lambda i: (i, 0)),
            pl.BlockSpec((BN, 16), lambda i: (i, 0)),
            blk((1, PERIODS)),
            blk((1, HID)),
            blk((1, HID)),
            blk((HID, HID)),
            blk((1, HID)),
            blk((1, HID)),
            blk((1, HID)),
            blk((HID, HID)),
            blk((1, HID)),
            blk((1, HID)),
            blk((1, 1)),
        ],
        out_specs=pl.BlockSpec((BN, 1), lambda i: (i, 0)),
        out_shape=jax.ShapeDtypeStruct((N, 1), jnp.float32),
    )(s0, s1, aux, att, Wcz, bcz, Wlz, blz, Wch, bch, Wlh, blh, wlin, blin)


def kernel(x, edge_index, edge_weight, att, W_cz, b_cz, W_lz, b_lz, W_cr,
           b_cr, W_lr, b_lr, W_ch, b_ch, W_lh, b_lh, W_lin, b_lin):
    src = edge_index[0].astype(jnp.int32)
    dst = edge_index[1].astype(jnp.int32)
    w = edge_weight.astype(jnp.float32)

    npadidx = (jnp.arange(EPAD - E, dtype=jnp.int32) * 173) % N
    src2d = jnp.concatenate([src, npadidx]).reshape(G, CHUNK)
    dst2d = jnp.concatenate([dst, npadidx]).reshape(G, CHUNK)
    w2d = jnp.concatenate(
        [w, jnp.zeros((EPAD - E,), jnp.float32)]).reshape(G, CHUNK)

    degp0, degp1 = _deg_call(dst2d, w2d)
    deg = degp0[:N] + degp1[:N] + 1.0
    dinv = lax.rsqrt(deg)

    aux = jnp.concatenate(
        [dinv[:, None] * x, dinv[:, None], jnp.zeros((N, 3), jnp.float32)],
        axis=1)

    sp0, sp1 = _scatter_call(src2d, dst2d, w2d, aux)

    return _gate_call(
        sp0, sp1, aux,
        att.reshape(1, PERIODS),
        W_cz.reshape(1, HID), b_cz.reshape(1, HID), W_lz[:HID],
        b_lz.reshape(1, HID),
        W_ch.reshape(1, HID), b_ch.reshape(1, HID), W_lh[:HID],
        b_lh.reshape(1, HID),
        W_lin.reshape(1, HID), b_lin.reshape(1, 1))


# speedup vs baseline: 477.5493x; 1.0885x over previous
import functools

import jax
import jax.numpy as jnp
from jax import lax
from jax.experimental import pallas as pl
from jax.experimental.pallas import tpu as pltpu
from jax.experimental.pallas import tpu_sc as plsc

N = 50000
E = 800000
PERIODS = 12
HID = 100

NC = 2
NS = 16
NW = NC * NS
CHUNK = 128
GTOT = E // CHUNK
MAIN = 192
GMAIN = NW * MAIN
NPAD = 50176
STRIPE = NPAD // NS
NBUF = 8


def _deg_body(ei, w, degp0, degp1, didx4, wrow4, zb, acc,
              se0, se1, se2, se3, ss0, ss1, ss2, ss3):
    c = lax.axis_index("c")
    s = lax.axis_index("s")
    wid = s * NC + c
    base = wid * MAIN
    se = [se0, se1, se2, se3]
    ss = [ss0, ss1, ss2, ss3]
    DB = 4

    def zero_body(i, carry):
        zb[pl.ds(i * 16, 16)] = jnp.zeros((16,), jnp.float32)
        return carry

    lax.fori_loop(0, STRIPE // 16, zero_body, 0)
    pltpu.sync_copy(zb, acc.at[pl.ds(s * STRIPE, STRIPE)])
    plsc.subcore_barrier()

    def issue_load(g, b):
        pltpu.async_copy(ei.at[1, pl.ds(g * CHUNK, CHUNK)], didx4.at[b],
                         se[b])
        pltpu.async_copy(w.at[pl.ds(g * CHUNK, CHUNK)], wrow4.at[b], se[b])

    def wait_load(g, b):
        pltpu.make_async_copy(ei.at[1, pl.ds(g * CHUNK, CHUNK)],
                              didx4.at[b], se[b]).wait()
        pltpu.make_async_copy(w.at[pl.ds(g * CHUNK, CHUNK)], wrow4.at[b],
                              se[b]).wait()

    def issue_scat(b):
        pltpu.async_copy(wrow4.at[b], acc.at[didx4.at[b]], ss[b], add=True)

    def wait_scat(b):
        pltpu.make_async_copy(
            wrow4.at[b], acc.at[didx4.at[b]], ss[b]).wait()

    issue_load(base + 0, 0)
    issue_load(base + 1, 1)
    for q in (0, 1):
        wait_load(base + q, q)
        issue_scat(q)
        issue_load(base + q + 2, (q + 2) % DB)

    def body(k, carry):
        for boff in range(DB):
            g = base + 2 + k * DB + boff
            b = (2 + boff) % DB
            wait_load(g, b)
            issue_scat(b)
            b2 = (b + 2) % DB
            wait_scat(b2)
            issue_load(g + 2, b2)
        return carry

    lax.fori_loop(0, (MAIN - 4) // DB, body, 0)

    for q in (MAIN - 2, MAIN - 1):
        b = q % DB
        wait_load(base + q, b)
        issue_scat(b)
        wait_scat((b + 2) % DB)
    wait_scat((MAIN - 2) % DB)
    wait_scat((MAIN - 1) % DB)

    def tail_body(k, carry):
        g = GMAIN + k * NW + wid
        pltpu.sync_copy(ei.at[1, pl.ds(g * CHUNK, CHUNK)], didx4.at[0])
        pltpu.sync_copy(w.at[pl.ds(g * CHUNK, CHUNK)], wrow4.at[0])
        pltpu.sync_copy(wrow4.at[0], acc.at[didx4.at[0]], add=True)
        return carry

    ntail = (GTOT - GMAIN) // NW + jnp.where(
        wid < (GTOT - GMAIN) % NW, 1, 0)
    lax.fori_loop(0, ntail, tail_body, 0)

    plsc.subcore_barrier()
    pltpu.sync_copy(acc.at[pl.ds(s * STRIPE, STRIPE)], zb)

    @pl.when(c == 0)
    def _():
        pltpu.sync_copy(zb, degp0.at[pl.ds(s * STRIPE, STRIPE)])

    @pl.when(c == 1)
    def _():
        pltpu.sync_copy(zb, degp1.at[pl.ds(s * STRIPE, STRIPE)])


def _scatter_body(ei, w, tab, sp0, sp1,
                  sidx8, didx8, wrow8, rows8, zb, acc, *sems):
    c = lax.axis_index("c")
    s = lax.axis_index("s")
    wid = s * NC + c
    base = wid * MAIN
    se = sems[0:NBUF]
    sg = sems[NBUF:2 * NBUF]
    ss = sems[2 * NBUF:3 * NBUF]

    def zero_body(i, carry):
        zb[i, :] = jnp.zeros((16,), jnp.float32)
        return carry

    lax.fori_loop(0, STRIPE, zero_body, 0)
    pltpu.sync_copy(zb, acc.at[pl.ds(s * STRIPE, STRIPE)])
    plsc.subcore_barrier()

    def issue_load(g, b):
        pltpu.async_copy(ei.at[0, pl.ds(g * CHUNK, CHUNK)], sidx8.at[b],
                         se[b])
        pltpu.async_copy(ei.at[1, pl.ds(g * CHUNK, CHUNK)], didx8.at[b],
                         se[b])
        pltpu.async_copy(w.at[pl.ds(g * CHUNK, CHUNK)], wrow8.at[b], se[b])

    def wait_load(g, b):
        pltpu.make_async_copy(ei.at[0, pl.ds(g * CHUNK, CHUNK)],
                              sidx8.at[b], se[b]).wait()
        pltpu.make_async_copy(ei.at[1, pl.ds(g * CHUNK, CHUNK)],
                              didx8.at[b], se[b]).wait()
        pltpu.make_async_copy(w.at[pl.ds(g * CHUNK, CHUNK)], wrow8.at[b],
                              se[b]).wait()

    def issue_gather(b):
        pltpu.async_copy(tab.at[sidx8.at[b]], rows8.at[b], sg[b])

    def wait_gather(b):
        pltpu.make_async_copy(tab.at[sidx8.at[b]], rows8.at[b],
                              sg[b]).wait()

    def issue_scat(b):
        pltpu.async_copy(rows8.at[b], acc.at[didx8.at[b]], ss[b], add=True)

    def wait_scat(b):
        pltpu.make_async_copy(
            rows8.at[b], acc.at[didx8.at[b]], ss[b]).wait()

    def scale(b):
        for blk in range(CHUNK // 16):
            w16 = wrow8[b, pl.ds(blk * 16, 16)]
            for l in range(16):
                e = blk * 16 + l
                rows8[b, e, :] = rows8[b, e, :] * w16[l]

    def step(g, sl, gather2, scatwait, load4):
        wait_gather(sl)
        scale(sl)
        issue_scat(sl)
        if gather2:
            sl2 = (sl + 2) % NBUF
            wait_load(g + 2, sl2)
            issue_gather(sl2)
        if scatwait:
            wait_scat((sl + 4) % NBUF)
        if load4:
            issue_load(g + 4, (sl + 4) % NBUF)

    for q in range(4):
        issue_load(base + q, q)
    for q in (0, 1):
        wait_load(base + q, q)
        issue_gather(q)
    for q in range(4):
        step(base + q, q, True, False, True)

    def body(k, carry):
        for boff in range(NBUF):
            g = base + 4 + k * NBUF + boff
            step(g, (4 + boff) % NBUF, True, True, True)
        return carry

    lax.fori_loop(0, (MAIN - 8) // NBUF, body, 0)

    for q in (MAIN - 4, MAIN - 3):
        step(base + q, q % NBUF, True, True, False)
    for q in (MAIN - 2, MAIN - 1):
        step(base + q, q % NBUF, False, True, False)
    for q in range(MAIN - 4, MAIN):
        wait_scat(q % NBUF)

    def tail_body(k, carry):
        g = GMAIN + k * NW + wid
        pltpu.sync_copy(ei.at[0, pl.ds(g * CHUNK, CHUNK)], sidx8.at[0])
        pltpu.sync_copy(ei.at[1, pl.ds(g * CHUNK, CHUNK)], didx8.at[0])
        pltpu.sync_copy(w.at[pl.ds(g * CHUNK, CHUNK)], wrow8.at[0])
        pltpu.async_copy(tab.at[sidx8.at[0]], rows8.at[0], sg[0]).wait()
        scale(0)
        pltpu.sync_copy(rows8.at[0], acc.at[didx8.at[0]], add=True)
        return carry

    ntail = (GTOT - GMAIN) // NW + jnp.where(
        wid < (GTOT - GMAIN) % NW, 1, 0)
    lax.fori_loop(0, ntail, tail_body, 0)

    plsc.subcore_barrier()
    pltpu.sync_copy(acc.at[pl.ds(s * STRIPE, STRIPE)], zb)

    @pl.when(c == 0)
    def _():
        pltpu.sync_copy(zb, sp0.at[pl.ds(s * STRIPE, STRIPE)])

    @pl.when(c == 1)
    def _():
        pltpu.sync_copy(zb, sp1.at[pl.ds(s * STRIPE, STRIPE)])


_deg_call = functools.partial(
    pl.kernel,
    out_type=(jax.ShapeDtypeStruct((NPAD,), jnp.float32),
              jax.ShapeDtypeStruct((NPAD,), jnp.float32)),
    mesh=plsc.VectorSubcoreMesh(core_axis_name="c", subcore_axis_name="s"),
    compiler_params=pltpu.CompilerParams(use_tc_tiling_on_sc=False),
    scratch_types=[
        pltpu.VMEM((4, CHUNK), jnp.int32),
        pltpu.VMEM((4, CHUNK), jnp.float32),
        pltpu.VMEM((STRIPE,), jnp.float32),
        pltpu.VMEM_SHARED((NPAD,), jnp.float32),
    ] + [pltpu.SemaphoreType.DMA] * 8,
)(_deg_body)

_scatter_call = functools.partial(
    pl.kernel,
    out_type=(jax.ShapeDtypeStruct((NPAD, 16), jnp.float32),
              jax.ShapeDtypeStruct((NPAD, 16), jnp.float32)),
    mesh=plsc.VectorSubcoreMesh(core_axis_name="c", subcore_axis_name="s"),
    compiler_params=pltpu.CompilerParams(use_tc_tiling_on_sc=False),
    scratch_types=[
        pltpu.VMEM((NBUF, CHUNK), jnp.int32),
        pltpu.VMEM((NBUF, CHUNK), jnp.int32),
        pltpu.VMEM((NBUF, CHUNK), jnp.float32),
        pltpu.VMEM((NBUF, CHUNK, 16), jnp.float32),
        pltpu.VMEM((STRIPE, 16), jnp.float32),
        pltpu.VMEM_SHARED((NPAD, 16), jnp.float32),
    ] + [pltpu.SemaphoreType.DMA] * 24,
)(_scatter_body)


BN = 5000


def _gate_body(s0, s1, aux, att, Wcz, bcz, Wlz, blz, Wch, bch, Wlh, blh,
               wlin, blin, out):
    a = jnp.exp(att[...] - jnp.max(att[...]))
    probs = a / jnp.sum(a)
    u_zn = -jnp.dot(Wcz[...], Wlz[...], preferred_element_type=jnp.float32)
    c_zn = -(jnp.dot(bcz[...], Wlz[...],
                     preferred_element_type=jnp.float32) + blz[...])
    u_h = jnp.dot(Wch[...], Wlh[...], preferred_element_type=jnp.float32)
    c_h = jnp.dot(bch[...], Wlh[...], preferred_element_type=jnp.float32) \
        + blh[...]
    dinv = aux[:, 12:13]
    gall = dinv * (s0[...] + s1[...] + aux[...])
    acc = jnp.zeros((BN, HID), jnp.float32)
    for t in range(PERIODS):
        g = gall[:, t:t + 1]
        zc = jax.nn.sigmoid(g * u_zn + c_zn)
        ht = jnp.tanh(g * u_h + c_h)
        acc = acc + probs[0, t] * (zc * ht)
    h = jnp.maximum(acc, 0.0)
    out[...] = jnp.sum(h * wlin[...], axis=1, keepdims=True) + blin[...]


def _gate_call(s0, s1, aux, att, Wcz, bcz, Wlz, blz, Wch, bch, Wlh, blh,
               wlin, blin):
    grid = (N // BN,)
    blk = lambda shape: pl.BlockSpec(shape, lambda i: (0,) * len(shape))
    return pl.pallas_call(
        _gate_body,
        grid=grid,
        in_specs=[
            pl.BlockSpec((BN, 16), lambda i: (i, 0)),
            pl.BlockSpec((BN, 16), lambda i: (i, 0)),
            pl.BlockSpec((BN, 16), lambda i: (i, 0)),
            blk((1, PERIODS)),
            blk((1, HID)),
            blk((1, HID)),
            blk((HID, HID)),
            blk((1, HID)),
            blk((1, HID)),
            blk((1, HID)),
            blk((HID, HID)),
            blk((1, HID)),
            blk((1, HID)),
            blk((1, 1)),
        ],
        out_specs=pl.BlockSpec((BN, 1), lambda i: (i, 0)),
        out_shape=jax.ShapeDtypeStruct((N, 1), jnp.float32),
    )(s0, s1, aux, att, Wcz, bcz, Wlz, blz, Wch, bch, Wlh, blh, wlin, blin)


def kernel(x, edge_index, edge_weight, att, W_cz, b_cz, W_lz, b_lz, W_cr,
           b_cr, W_lr, b_lr, W_ch, b_ch, W_lh, b_lh, W_lin, b_lin):
    ei = edge_index.astype(jnp.int32)
    w = edge_weight.astype(jnp.float32)

    degp0, degp1 = _deg_call(ei, w)
    deg = degp0[:N] + degp1[:N] + 1.0
    dinv = lax.rsqrt(deg)

    aux = jnp.concatenate(
        [dinv[:, None] * x, dinv[:, None], jnp.zeros((N, 3), jnp.float32)],
        axis=1)

    sp0, sp1 = _scatter_call(ei, w, aux)

    return _gate_call(
        sp0, sp1, aux,
        att.reshape(1, PERIODS),
        W_cz.reshape(1, HID), b_cz.reshape(1, HID), W_lz[:HID],
        b_lz.reshape(1, HID),
        W_ch.reshape(1, HID), b_ch.reshape(1, HID), W_lh[:HID],
        b_lh.reshape(1, HID),
        W_lin.reshape(1, HID), b_lin.reshape(1, 1))
